# Initial kernel scaffold; baseline (speedup 1.0000x reference)
import jax
import jax.numpy as jnp
from jax.experimental import pallas as pl


def kernel(x, edge_index, Wg0, bg0, W0, b0, Wr0, Wg1, bg1, W1, b1, Wr1, Wl2, bl2, Wr2):
    raise NotImplementedError("write your pallas kernel here")


# speedup vs baseline: 6.7269x; 6.7269x over previous
import functools

import jax
import jax.numpy as jnp
from jax import lax
from jax.experimental import pallas as pl
from jax.experimental.pallas import tpu as pltpu
from jax.experimental.pallas import tpu_sc as plsc

N = 10000
E = 160000
DIN = 256
DH = 256
DOUT = 128
NE = 8

NSC = 2
NTILES = 16
CHUNK = 80
NBUF = 5
EDGES_PER_TILE = E // NTILES
CHUNKS_PER_TILE = EDGES_PER_TILE // CHUNK
ROWS_MAIN = 624
TAIL0 = NTILES * ROWS_MAIN
TAILN = N - TAIL0

_f32 = jnp.float32


@functools.lru_cache(maxsize=None)
def _build_segsum(qw: int, with_counts: bool, interpret: bool = False):
    mesh = plsc.VectorSubcoreMesh(core_axis_name="c", subcore_axis_name="s",
                                  num_cores=NSC, num_subcores=NTILES)
    out_type = [jax.ShapeDtypeStruct((N, qw), _f32) for _ in range(4)]
    if with_counts:
        out_type.append(jax.ShapeDtypeStruct((N, 16), _f32))
    scratch = [
        pltpu.VMEM((CHUNKS_PER_TILE, CHUNK), jnp.int32),
        pltpu.VMEM((CHUNKS_PER_TILE, CHUNK), jnp.int32),
        pltpu.VMEM((NBUF, CHUNK, qw), _f32),
        pltpu.VMEM((CHUNK, 16), _f32),
        pltpu.VMEM_SHARED((N, qw), _f32),
    ]
    if with_counts:
        scratch.append(pltpu.VMEM_SHARED((N, 16), _f32))
    scratch += [pltpu.SemaphoreType.DMA] * NBUF

    def body(xq0, xq1, xq2, xq3, src_hbm, dst_hbm, z_hbm, z16_hbm, *rest):
        xs = [xq0, xq1, xq2, xq3]
        outs = list(rest[:4])
        rest = rest[4:]
        if with_counts:
            cnt_out = rest[0]
            rest = rest[1:]
        src_v, dst_v, rows_v, ones_v, acc = rest[:5]
        rest = rest[5:]
        if with_counts:
            cnt_acc = rest[0]
            rest = rest[1:]
        sems = rest
        cid = lax.axis_index("c")
        tid = lax.axis_index("s")

        row0 = tid * ROWS_MAIN
        last = tid == NTILES - 1

        def sliced_copy(src, dst):
            pltpu.sync_copy(src.at[pl.ds(row0, ROWS_MAIN)],
                            dst.at[pl.ds(row0, ROWS_MAIN)])

            @pl.when(last)
            def _():
                pltpu.sync_copy(src.at[pl.ds(TAIL0, TAILN)],
                                dst.at[pl.ds(TAIL0, TAILN)])

        def start_gather(x_hbm, j, b):
            return pltpu.async_copy(x_hbm.at[src_v.at[j]], rows_v.at[b],
                                    sems[b])

        def wait_gather(x_hbm, j, b):
            pltpu.make_async_copy(x_hbm.at[src_v.at[j]], rows_v.at[b],
                                  sems[b]).wait()

        for c in range(NSC):
            @pl.when(cid == c)
            def _(c=c):
                pltpu.sync_copy(src_hbm.at[tid], src_v)
                pltpu.sync_copy(dst_hbm.at[tid], dst_v)
                if with_counts and c == 0:
                    for r in range(CHUNK):
                        ones_v[r] = jnp.full((16,), 1.0, _f32)
                for p in range(2):
                    q = 2 * c + p
                    do_cnt = with_counts and q == 0
                    x_hbm = xs[q]
                    sliced_copy(z_hbm, acc)
                    if do_cnt:
                        sliced_copy(z16_hbm, cnt_acc)
                    plsc.subcore_barrier()
                    for b in range(NBUF):
                        start_gather(x_hbm, b, b)

                    def step(i, _, x_hbm=x_hbm, do_cnt=do_cnt):
                        j0 = i * NBUF
                        for b in range(NBUF):
                            j = j0 + b
                            wait_gather(x_hbm, j, b)
                            pltpu.sync_copy(rows_v.at[b],
                                            acc.at[dst_v.at[j]], add=True)
                            if do_cnt:
                                pltpu.sync_copy(ones_v,
                                                cnt_acc.at[dst_v.at[j]],
                                                add=True)
                            nj = j + NBUF

                            @pl.when(nj < CHUNKS_PER_TILE)
                            def _(b=b, nj=nj, x_hbm=x_hbm):
                                start_gather(x_hbm, nj, b)
                        return 0

                    lax.fori_loop(0, CHUNKS_PER_TILE // NBUF, step, 0)
                    plsc.subcore_barrier()
                    sliced_copy(acc, outs[q])
                    if do_cnt:
                        sliced_copy(cnt_acc, cnt_out)

    return pl.kernel(body, out_type=tuple(out_type), mesh=mesh,
                     scratch_types=tuple(scratch), interpret=interpret,
                     compiler_params=pltpu.CompilerParams(
                         use_tc_tiling_on_sc=False))


def _segsum(xqs, src3, dst3, qw, with_counts, z16):
    zq = jnp.zeros((N, qw), _f32)
    return _build_segsum(qw, with_counts)(*xqs, src3, dst3, zq, z16)


TB = 1000


def _moe_body(sq_refs, cnt_ref, xq_refs, Wg_ref, bg_ref, W_ref, b_ref,
              Wr_ref, Wl2_ref, outq_refs, y2q_refs, gs_ref):
    i = pl.program_id(0)
    s = jnp.concatenate([r[...] for r in sq_refs], axis=1)
    cnt = jnp.maximum(cnt_ref[...][:, 0:1], 1.0)
    h = s / cnt
    logits = jnp.dot(h, Wg_ref[...], preferred_element_type=_f32) + bg_ref[...]
    m = jnp.max(logits, axis=1, keepdims=True)
    eg = jnp.exp(logits - m)
    g = eg / jnp.sum(eg, axis=1, keepdims=True)
    gm = jnp.mean(g, axis=1, keepdims=True)
    stds = jnp.sqrt(jnp.sum((g - gm) ** 2, axis=1, keepdims=True) / (NE - 1))

    @pl.when(i == 0)
    def _():
        gs_ref[...] = jnp.zeros((1, 1), _f32)

    gs_ref[...] = gs_ref[...] + jnp.sum(stds).reshape(1, 1)

    x = jnp.concatenate([r[...] for r in xq_refs], axis=1)
    acc = jnp.dot(x, Wr_ref[...], preferred_element_type=_f32)
    found = jnp.zeros((TB, 1), jnp.bool_)
    for e in range(NE):
        is_e = jnp.logical_and(logits[:, e:e + 1] == m,
                               jnp.logical_not(found))
        found = jnp.logical_or(found, is_e)
        mask = is_e.astype(_f32)
        acc += jnp.dot(h * mask, W_ref[e],
                       preferred_element_type=_f32) + mask * b_ref[e:e + 1, :]
    xn = jnp.maximum(acc, 0.0)
    for q in range(4):
        outq_refs[q][...] = xn[:, 64 * q:64 * (q + 1)]
    if y2q_refs is not None:
        y2 = jnp.dot(xn, Wl2_ref[...], preferred_element_type=_f32)
        for q in range(4):
            y2q_refs[q][...] = y2[:, 32 * q:32 * (q + 1)]


@functools.lru_cache(maxsize=None)
def _make_moe_layer(with_y2: bool, interpret: bool = False):
    def body(*refs):
        sq = refs[0:4]
        cnt = refs[4]
        xq = refs[5:9]
        Wg, bg, W, b, Wr = refs[9:14]
        k = 14
        Wl2 = refs[k] if with_y2 else None
        k += 1 if with_y2 else 0
        outq = refs[k:k + 4]
        k += 4
        y2q = refs[k:k + 4] if with_y2 else None
        k += 4 if with_y2 else 0
        gs = refs[k]
        _moe_body(sq, cnt, xq, Wg, bg, W, b, Wr, Wl2, outq, y2q, gs)

    grid = (N // TB,)
    tok = lambda w: pl.BlockSpec((TB, w), lambda i: (i, 0))
    full = lambda *shape: pl.BlockSpec(shape, lambda i: tuple(0 for _ in shape))
    in_specs = [tok(64)] * 4 + [tok(16)] + [tok(64)] * 4 + [
        full(DH, NE), full(1, NE), full(NE, DH, DH), full(NE, DH),
        full(DH, DH)]
    out_shapes = [jax.ShapeDtypeStruct((N, 64), _f32) for _ in range(4)]
    out_specs = [tok(64)] * 4
    if with_y2:
        in_specs.append(full(DH, DOUT))
        out_shapes += [jax.ShapeDtypeStruct((N, 32), _f32) for _ in range(4)]
        out_specs += [tok(32)] * 4
    out_shapes.append(jax.ShapeDtypeStruct((1, 1), _f32))
    out_specs.append(pl.BlockSpec((1, 1), lambda i: (0, 0)))
    return pl.pallas_call(
        body, grid=grid, in_specs=in_specs, out_specs=out_specs,
        out_shape=tuple(out_shapes), interpret=interpret)


def _final_body(*refs):
    s2q = refs[0:4]
    cnt_ref = refs[4]
    x2q = refs[5:9]
    Wr2_ref, bl2_ref, out_ref = refs[9:12]
    s2 = jnp.concatenate([r[...] for r in s2q], axis=1)
    cnt = jnp.maximum(cnt_ref[...][:, 0:1], 1.0)
    x2 = jnp.concatenate([r[...] for r in x2q], axis=1)
    out_ref[...] = (s2 / cnt + bl2_ref[...]
                    + jnp.dot(x2, Wr2_ref[...], preferred_element_type=_f32))


@functools.lru_cache(maxsize=None)
def _make_final(interpret: bool = False):
    return pl.pallas_call(
        _final_body, grid=(N // TB,),
        in_specs=[pl.BlockSpec((TB, 32), lambda i: (i, 0))] * 4
        + [pl.BlockSpec((TB, 16), lambda i: (i, 0))]
        + [pl.BlockSpec((TB, 64), lambda i: (i, 0))] * 4
        + [pl.BlockSpec((DH, DOUT), lambda i: (0, 0)),
           pl.BlockSpec((1, DOUT), lambda i: (0, 0))],
        out_specs=pl.BlockSpec((TB, DOUT), lambda i: (i, 0)),
        out_shape=jax.ShapeDtypeStruct((N, DOUT), _f32),
        interpret=interpret)


def kernel(x, edge_index, Wg0, bg0, W0, b0, Wr0, Wg1, bg1, W1, b1, Wr1,
           Wl2, bl2, Wr2):
    xq = tuple(x[:, 64 * q:64 * (q + 1)] for q in range(4))
    src3 = edge_index[0].reshape(NTILES, CHUNKS_PER_TILE, CHUNK)
    dst3 = edge_index[1].reshape(NTILES, CHUNKS_PER_TILE, CHUNK)
    z16 = jnp.zeros((N, 16), _f32)

    *s0q, cnt16 = _segsum(xq, src3, dst3, 64, True, z16)
    r0 = _make_moe_layer(False)(*s0q, cnt16, *xq, Wg0, bg0.reshape(1, NE),
                                W0, b0, Wr0)
    x1q, gs0 = r0[:4], r0[4]
    s1q = _segsum(x1q, src3, dst3, 64, False, z16)
    r1 = _make_moe_layer(True)(*s1q, cnt16, *x1q, Wg1, bg1.reshape(1, NE),
                               W1, b1, Wr1, Wl2)
    x2q, y2q, gs1 = r1[:4], r1[4:8], r1[8]
    s2q = _segsum(y2q, src3, dst3, 32, False, z16)
    out = _make_final()(*s2q, cnt16, *x2q, Wr2, bl2.reshape(1, DOUT))
    gstd = (gs0[0, 0] + gs1[0, 0]) / (2.0 * N)
    return out, gstd
